# baseline (speedup 1.0000x reference)
import functools

import jax
import jax.numpy as jnp
from jax import lax
from jax.experimental import pallas as pl
from jax.experimental.pallas import tpu as pltpu
from jax.experimental.pallas import tpu_sc as plsc

D = 32
V = 1000000
B = 16384
H = 50
NC, NS = 2, 16
NW = NC * NS
HPC = H // NC
BPT = B // NS
CHUNK = 128
NCHUNK = BPT // CHUNK
L = 16
ACC_ROWS = 32

NBLK = 7813
NFULL = NBLK - 1
TAIL_W = V - NFULL * 128
TP = 129
VPAD = NBLK * CHUNK

_mesh = plsc.VectorSubcoreMesh(core_axis_name="c", subcore_axis_name="s")


@functools.partial(
    pl.kernel,
    mesh=_mesh,
    compiler_params=pltpu.CompilerParams(
        use_tc_tiling_on_sc=True, needs_layout_passes=False
    ),
    out_type=jax.ShapeDtypeStruct((NBLK, D, CHUNK), jnp.float32),
    scratch_types=[
        pltpu.VMEM((2, D, TP), jnp.float32),
        pltpu.VMEM((2, D, CHUNK), jnp.float32),
        pltpu.SemaphoreType.DMA((2,)),
        pltpu.SemaphoreType.DMA((2,)),
    ],
)
def _repack_sc(tt_hbm, tail_hbm, out_hbm, tbuf_v, rows_v, isems, osems):
    c = lax.axis_index("c")
    s = lax.axis_index("s")
    w = s * NC + c
    lanes = jnp.arange(L, dtype=jnp.int32)
    lanes_hi = lanes + L

    start = w * 244 + jnp.maximum(w - 27, 0)
    nfull = jnp.where(w >= 27, 245, 244) - jnp.where(w == 31, 1, 0)

    def dma_in(blk, p):
        for db in range(4):
            yield pltpu.make_async_copy(
                tt_hbm.at[pl.ds(db * 8, 8), pl.ds(blk * CHUNK, CHUNK)],
                tbuf_v.at[p, pl.ds(db * 8, 8), pl.ds(0, CHUNK)],
                isems.at[p],
            )

    def dma_out(blk, p):
        return pltpu.make_async_copy(rows_v.at[p], out_hbm.at[blk], osems.at[p])

    def start_in(blk, p):
        for cp in dma_in(blk, p):
            cp.start()

    def wait_in(p):
        for cp in dma_in(0, p):
            cp.wait()

    one = jnp.ones((L,), jnp.int32)

    def transpose_block(p):
        buf = tbuf_v.at[p]
        col = jnp.zeros((L,), jnp.int32)
        for vi in range(D):
            for u in range(4):
                lo = plsc.load_gather(buf, [lanes, col])
                hi = plsc.load_gather(buf, [lanes_hi, col])
                rows_v[p, vi, pl.ds(u * D, L)] = lo
                rows_v[p, vi, pl.ds(u * D + L, L)] = hi
                col = col + one

    start_in(start, 0)

    def g_body(gg, _):
        for p in range(2):
            g = gg * 2 + p

            @pl.when(g < nfull)
            def _():
                blk = start + g

                @pl.when(g + 1 < nfull)
                def _():
                    start_in(start + g + 1, 1 - p)

                wait_in(p)

                @pl.when(g >= 2)
                def _():
                    dma_out(0, p).wait()

                transpose_block(p)
                dma_out(blk, p).start()

        return 0

    lax.fori_loop(0, 123, g_body, 0)

    @pl.when(nfull >= 1)
    def _():
        dma_out(0, (nfull - 1) % 2).wait()

    @pl.when(nfull >= 2)
    def _():
        dma_out(0, (nfull - 2) % 2).wait()

    @pl.when(w == NW - 1)
    def _():
        for db in range(4):
            pltpu.sync_copy(
                tail_hbm.at[pl.ds(db * 8, 8), pl.ds(0, CHUNK)],
                tbuf_v.at[0, pl.ds(db * 8, 8), pl.ds(0, CHUNK)],
            )

        def vi_body(vi, _):
            v4 = vi * 4
            for u in range(4):
                col = jnp.full((L,), v4 + u, jnp.int32)
                lo = plsc.load_gather(tbuf_v.at[0], [lanes, col])
                hi = plsc.load_gather(tbuf_v.at[0], [lanes_hi, col])
                rows_v[0, vi, pl.ds(u * D, L)] = lo
                rows_v[0, vi, pl.ds(u * D + L, L)] = hi
            return 0

        lax.fori_loop(0, TAIL_W // 4, vi_body, 0)
        pltpu.sync_copy(
            rows_v.at[0, pl.ds(0, TAIL_W // 4)],
            out_hbm.at[NFULL, pl.ds(0, TAIL_W // 4)],
        )


@functools.partial(
    pl.kernel,
    mesh=_mesh,
    compiler_params=pltpu.CompilerParams(
        use_tc_tiling_on_sc=False, needs_layout_passes=False
    ),
    out_type=jax.ShapeDtypeStruct((H, D), jnp.float32),
    scratch_types=[
        pltpu.VMEM((BPT, H), jnp.int32),
        pltpu.VMEM((BPT,), jnp.int32),
        pltpu.VMEM((2, CHUNK, D), jnp.float32),
        pltpu.VMEM((ACC_ROWS, D), jnp.float32),
        pltpu.VMEM((ACC_ROWS,), jnp.int32),
        pltpu.VMEM((D,), jnp.float32),
        pltpu.VMEM_SHARED((ACC_ROWS, D), jnp.float32),
        pltpu.SemaphoreType.DMA((2,)),
    ],
)
def _cbow_sc(idx_hbm, table_hbm, out_hbm, idxblk_v, list_v, rows_v, acc_v,
             rowids_v, st_v, shared_acc, sems):
    c = lax.axis_index("c")
    s = lax.axis_index("s")
    lanes = jnp.arange(L, dtype=jnp.int32)

    def gather(k, p):
        return pltpu.make_async_copy(
            table_hbm.at[list_v.at[pl.ds(k * CHUNK, CHUNK)]],
            rows_v.at[p],
            sems.at[p],
        )

    def accum_chunk(buf, accs):
        def row_body(r, a):
            a0, a1, a2, a3, a4, a5, a6, a7 = a
            base = r * 4
            a0 = a0 + buf[base, pl.ds(0, L)]
            a1 = a1 + buf[base, pl.ds(L, L)]
            a2 = a2 + buf[base + 1, pl.ds(0, L)]
            a3 = a3 + buf[base + 1, pl.ds(L, L)]
            a4 = a4 + buf[base + 2, pl.ds(0, L)]
            a5 = a5 + buf[base + 2, pl.ds(L, L)]
            a6 = a6 + buf[base + 3, pl.ds(0, L)]
            a7 = a7 + buf[base + 3, pl.ds(L, L)]
            return (a0, a1, a2, a3, a4, a5, a6, a7)

        return lax.fori_loop(0, CHUNK // 4, row_body, accs)

    pltpu.sync_copy(idx_hbm.at[pl.ds(s * BPT, BPT)], idxblk_v)

    rowids_v[pl.ds(0, L)] = lanes
    rowids_v[pl.ds(L, L)] = lanes + L

    def j_body(j, _):
        h = 2 * j + c

        def compact_body(v, _):
            rows = lanes + v * L
            cols = jnp.broadcast_to(h, (L,)).astype(jnp.int32)
            list_v[pl.ds(v * L, L)] = plsc.load_gather(idxblk_v, [rows, cols])
            return 0

        lax.fori_loop(0, BPT // L, compact_body, 0)

        gather(0, 0).start()
        zero = jnp.zeros((L,), jnp.float32)

        def g_body(g, accs):
            k0 = g * 2
            gather(k0 + 1, 1).start()
            gather(0, 0).wait()
            accs = accum_chunk(rows_v.at[0], accs)

            @pl.when(k0 + 2 < NCHUNK)
            def _():
                gather(k0 + 2, 0).start()

            gather(0, 1).wait()
            return accum_chunk(rows_v.at[1], accs)

        accs = lax.fori_loop(0, NCHUNK // 2, g_body, (zero,) * 8)
        acc_v[j, pl.ds(0, L)] = (accs[0] + accs[2]) + (accs[4] + accs[6])
        acc_v[j, pl.ds(L, L)] = (accs[1] + accs[3]) + (accs[5] + accs[7])
        return 0

    lax.fori_loop(0, HPC, j_body, 0)

    @pl.when(s == 0)
    def _():
        pltpu.sync_copy(acc_v, shared_acc)

    plsc.subcore_barrier()

    @pl.when(s != 0)
    def _():
        pltpu.sync_copy(acc_v, shared_acc.at[rowids_v], add=True)

    plsc.subcore_barrier()

    scale = jnp.float32(1.0 / B)

    def writeout(hl):
        pltpu.sync_copy(shared_acc.at[hl], st_v)
        st_v[pl.ds(0, L)] = st_v[pl.ds(0, L)] * scale
        st_v[pl.ds(L, L)] = st_v[pl.ds(L, L)] * scale
        pltpu.sync_copy(st_v, out_hbm.at[2 * hl + c])

    writeout(s)

    @pl.when(s + NS < HPC)
    def _():
        writeout(s + NS)


def kernel(input, table):
    tail = jnp.pad(jnp.transpose(table[NFULL * CHUNK :, :]),
                   ((0, 0), (0, CHUNK - TAIL_W)))
    packed = _repack_sc(jnp.transpose(table), tail)
    table_rm = jnp.reshape(packed, (VPAD, D))
    return _cbow_sc(input.astype(jnp.int32), table_rm)


# speedup vs baseline: 1.3627x; 1.3627x over previous
import functools

import jax
import jax.numpy as jnp
from jax import lax
from jax.experimental import pallas as pl
from jax.experimental.pallas import tpu as pltpu
from jax.experimental.pallas import tpu_sc as plsc

D = 32
B = 16384
H = 50
NC, NS = 2, 16
HPC = H // NC
BPT = B // NS
CHUNK = 128
NCHUNK = BPT // CHUNK
L = 16
ACC_ROWS = 32

_mesh = plsc.VectorSubcoreMesh(core_axis_name="c", subcore_axis_name="s")


@functools.partial(
    pl.kernel,
    mesh=_mesh,
    compiler_params=pltpu.CompilerParams(
        use_tc_tiling_on_sc=False, needs_layout_passes=False
    ),
    out_type=jax.ShapeDtypeStruct((H, D), jnp.float32),
    scratch_types=[
        pltpu.VMEM((BPT, H), jnp.int32),
        pltpu.VMEM((BPT,), jnp.int32),
        pltpu.VMEM((2, CHUNK, D), jnp.float32),
        pltpu.VMEM((ACC_ROWS, D), jnp.float32),
        pltpu.VMEM((ACC_ROWS,), jnp.int32),
        pltpu.VMEM((D,), jnp.float32),
        pltpu.VMEM_SHARED((ACC_ROWS, D), jnp.float32),
        pltpu.SemaphoreType.DMA((2,)),
    ],
)
def _cbow_sc(idx_hbm, table_hbm, out_hbm, idxblk_v, list_v, rows_v, acc_v,
             rowids_v, st_v, shared_acc, sems):
    c = lax.axis_index("c")
    s = lax.axis_index("s")
    lanes = jnp.arange(L, dtype=jnp.int32)

    def gather(k, p):
        return pltpu.make_async_copy(
            table_hbm.at[list_v.at[pl.ds(k * CHUNK, CHUNK)]],
            rows_v.at[p],
            sems.at[p],
        )

    def accum_chunk(buf, accs):
        def row_body(r, a):
            a0, a1, a2, a3, a4, a5, a6, a7 = a
            base = r * 4
            a0 = a0 + buf[base, pl.ds(0, L)]
            a1 = a1 + buf[base, pl.ds(L, L)]
            a2 = a2 + buf[base + 1, pl.ds(0, L)]
            a3 = a3 + buf[base + 1, pl.ds(L, L)]
            a4 = a4 + buf[base + 2, pl.ds(0, L)]
            a5 = a5 + buf[base + 2, pl.ds(L, L)]
            a6 = a6 + buf[base + 3, pl.ds(0, L)]
            a7 = a7 + buf[base + 3, pl.ds(L, L)]
            return (a0, a1, a2, a3, a4, a5, a6, a7)

        return lax.fori_loop(0, CHUNK // 4, row_body, accs)

    pltpu.sync_copy(idx_hbm.at[pl.ds(s * BPT, BPT)], idxblk_v)

    rowids_v[pl.ds(0, L)] = lanes
    rowids_v[pl.ds(L, L)] = lanes + L

    def j_body(j, _):
        h = 2 * j + c

        def compact_body(v, _):
            rows = lanes + v * L
            cols = jnp.broadcast_to(h, (L,)).astype(jnp.int32)
            list_v[pl.ds(v * L, L)] = plsc.load_gather(idxblk_v, [rows, cols])
            return 0

        lax.fori_loop(0, BPT // L, compact_body, 0)

        gather(0, 0).start()
        zero = jnp.zeros((L,), jnp.float32)

        def g_body(g, accs):
            k0 = g * 2
            gather(k0 + 1, 1).start()
            gather(0, 0).wait()
            accs = accum_chunk(rows_v.at[0], accs)

            @pl.when(k0 + 2 < NCHUNK)
            def _():
                gather(k0 + 2, 0).start()

            gather(0, 1).wait()
            return accum_chunk(rows_v.at[1], accs)

        accs = lax.fori_loop(0, NCHUNK // 2, g_body, (zero,) * 8)
        acc_v[j, pl.ds(0, L)] = (accs[0] + accs[2]) + (accs[4] + accs[6])
        acc_v[j, pl.ds(L, L)] = (accs[1] + accs[3]) + (accs[5] + accs[7])
        return 0

    lax.fori_loop(0, HPC, j_body, 0)

    @pl.when(s == 0)
    def _():
        pltpu.sync_copy(acc_v, shared_acc)

    plsc.subcore_barrier()

    @pl.when(s != 0)
    def _():
        pltpu.sync_copy(acc_v, shared_acc.at[rowids_v], add=True)

    plsc.subcore_barrier()

    scale = jnp.float32(1.0 / B)

    def writeout(hl):
        pltpu.sync_copy(shared_acc.at[hl], st_v)
        st_v[pl.ds(0, L)] = st_v[pl.ds(0, L)] * scale
        st_v[pl.ds(L, L)] = st_v[pl.ds(L, L)] * scale
        pltpu.sync_copy(st_v, out_hbm.at[2 * hl + c])

    writeout(s)

    @pl.when(s + NS < HPC)
    def _():
        writeout(s + NS)


def kernel(input, table):
    return _cbow_sc(input.astype(jnp.int32), table)
